# baseline (speedup 1.0000x reference)
import functools

import jax
import jax.numpy as jnp
from jax import lax
from jax.experimental import pallas as pl
from jax.experimental.pallas import tpu as pltpu

_B, _DIN, _H, _DC, _K = 16384, 512, 256, 32, 1024
_PREC = None
_BS = 1024
_NB = _B // _BS


def _vqvae_body(x_ref, ew1, eb1, ew2, eb2, ew3, eb3, cb, cbt,
                dw1, db1, dw2, db2, dw3, db3,
                dec_ref, zq_ref, loss_ref):
    f32 = jnp.float32
    x = x_ref[...]
    h = jnp.maximum(jnp.dot(x, ew1[...], precision=_PREC, preferred_element_type=f32) + eb1[...], 0.0)
    h = jnp.maximum(jnp.dot(h, ew2[...], precision=_PREC, preferred_element_type=f32) + eb2[...], 0.0)
    z = jnp.dot(h, ew3[...], precision=_PREC, preferred_element_type=f32) + eb3[...]

    cbt_v = cbt[...]
    csq = jnp.sum(cbt_v * cbt_v, axis=0, keepdims=True)
    zsq = jnp.sum(z * z, axis=1, keepdims=True)
    d = (zsq - 2.0 * jnp.dot(z, cbt_v, precision=_PREC, preferred_element_type=f32)) + csq

    dmin = jnp.min(d, axis=1, keepdims=True)
    iota_k = lax.broadcasted_iota(jnp.int32, (_BS, _K), 1).astype(f32)
    idx = jnp.min(jnp.where(d == dmin, iota_k, float(_K)), axis=1, keepdims=True)
    onehot = (iota_k == idx).astype(f32)
    e = jnp.dot(onehot, cb[...], precision=lax.Precision.HIGHEST,
                preferred_element_type=f32)

    zq = z + (e - z)
    zq_ref[...] = zq
    diff = z - e
    loss_ref[...] = jnp.full((1, 8, 128), jnp.sum(diff * diff), dtype=f32)

    g = jnp.maximum(jnp.dot(e, dw1[...], precision=_PREC, preferred_element_type=f32) + db1[...], 0.0)
    g = jnp.maximum(jnp.dot(g, dw2[...], precision=_PREC, preferred_element_type=f32) + db2[...], 0.0)
    dec_ref[...] = jnp.dot(g, dw3[...], precision=_PREC, preferred_element_type=f32) + db3[...]


def _full(shape):
    return pl.BlockSpec(shape, lambda i: (0,) * len(shape))


@jax.jit
def _vqvae_fused(x, enc_w1, enc_b1, enc_w2, enc_b2, enc_w3, enc_b3,
                 codebook, cb_t, dec_w1, dec_b1, dec_w2, dec_b2, dec_w3, dec_b3):
    dec, zq, loss_parts = pl.pallas_call(
        _vqvae_body,
        grid=(_NB,),
        in_specs=[
            pl.BlockSpec((_BS, _DIN), lambda i: (i, 0)),
            _full((_DIN, _H)), _full((1, _H)),
            _full((_H, _H)), _full((1, _H)),
            _full((_H, _DC)), _full((1, _DC)),
            _full((_K, _DC)), _full((_DC, _K)),
            _full((_DC, _H)), _full((1, _H)),
            _full((_H, _H)), _full((1, _H)),
            _full((_H, _DIN)), _full((1, _DIN)),
        ],
        out_specs=[
            pl.BlockSpec((_BS, _DIN), lambda i: (i, 0)),
            pl.BlockSpec((_BS, _DC), lambda i: (i, 0)),
            pl.BlockSpec((1, 8, 128), lambda i: (i, 0, 0)),
        ],
        out_shape=[
            jax.ShapeDtypeStruct((_B, _DIN), jnp.float32),
            jax.ShapeDtypeStruct((_B, _DC), jnp.float32),
            jax.ShapeDtypeStruct((_NB, 8, 128), jnp.float32),
        ],
        compiler_params=pltpu.CompilerParams(
            dimension_semantics=("arbitrary",),
        ),
    )(x, enc_w1, enc_b1, enc_w2, enc_b2, enc_w3, enc_b3, codebook, cb_t,
      dec_w1, dec_b1, dec_w2, dec_b2, dec_w3, dec_b3)
    loss = jnp.sum(loss_parts[:, 0, 0]) / (_B * _DC)
    return dec, zq, loss, loss


def kernel(x, enc_w1, enc_b1, enc_w2, enc_b2, enc_w3, enc_b3, codebook,
           dec_w1, dec_b1, dec_w2, dec_b2, dec_w3, dec_b3):
    return _vqvae_fused(
        x, enc_w1, enc_b1.reshape(1, -1), enc_w2, enc_b2.reshape(1, -1),
        enc_w3, enc_b3.reshape(1, -1), codebook, codebook.T,
        dec_w1, dec_b1.reshape(1, -1), dec_w2, dec_b2.reshape(1, -1),
        dec_w3, dec_b3.reshape(1, -1))


# speedup vs baseline: 1.2147x; 1.2147x over previous
import functools

import jax
import jax.numpy as jnp
from jax import lax
from jax.experimental import pallas as pl
from jax.experimental.pallas import tpu as pltpu

_B, _DIN, _H, _DC, _K = 16384, 512, 256, 32, 1024
_PREC = None
_BS = 1024
_NB = _B // _BS


def _vqvae_body(x_ref, ew1, eb1, ew2, eb2, ew3, eb3, cb_hi, cb_mid, cb_lo, cbt,
                dw1, db1, dw2, db2, dw3, db3,
                dec_ref, zq_ref, loss_ref):
    f32 = jnp.float32
    x = x_ref[...]
    h = jnp.maximum(jnp.dot(x, ew1[...], precision=_PREC, preferred_element_type=f32) + eb1[...], 0.0)
    h = jnp.maximum(jnp.dot(h, ew2[...], precision=_PREC, preferred_element_type=f32) + eb2[...], 0.0)
    z = jnp.dot(h, ew3[...], precision=_PREC, preferred_element_type=f32) + eb3[...]

    cbt_v = cbt[...]
    csq = jnp.sum(cbt_v * cbt_v, axis=0, keepdims=True)
    zsq = jnp.sum(z * z, axis=1, keepdims=True)
    d = (zsq - 2.0 * jnp.dot(z, cbt_v, precision=_PREC, preferred_element_type=f32)) + csq

    dmin = jnp.min(d, axis=1, keepdims=True)
    iota_k = lax.broadcasted_iota(jnp.int32, (_BS, _K), 1).astype(f32)
    idx = jnp.min(jnp.where(d == dmin, iota_k, float(_K)), axis=1, keepdims=True)
    onehot = (iota_k == idx).astype(f32)
    e = ((jnp.dot(onehot, cb_hi[...], precision=_PREC, preferred_element_type=f32)
          + jnp.dot(onehot, cb_mid[...], precision=_PREC, preferred_element_type=f32))
         + jnp.dot(onehot, cb_lo[...], precision=_PREC, preferred_element_type=f32))

    zq = z + (e - z)
    zq_ref[...] = zq
    diff = z - e
    loss_ref[...] = jnp.full((1, 8, 128), jnp.sum(diff * diff), dtype=f32)

    g = jnp.maximum(jnp.dot(e, dw1[...], precision=_PREC, preferred_element_type=f32) + db1[...], 0.0)
    g = jnp.maximum(jnp.dot(g, dw2[...], precision=_PREC, preferred_element_type=f32) + db2[...], 0.0)
    dec_ref[...] = jnp.dot(g, dw3[...], precision=_PREC, preferred_element_type=f32) + db3[...]


def _full(shape):
    return pl.BlockSpec(shape, lambda i: (0,) * len(shape))


@jax.jit
def _vqvae_fused(x, enc_w1, enc_b1, enc_w2, enc_b2, enc_w3, enc_b3,
                 codebook, dec_w1, dec_b1, dec_w2, dec_b2, dec_w3, dec_b3):
    bf16, f32 = jnp.bfloat16, jnp.float32
    cb_hi = codebook.astype(bf16).astype(f32)
    rem = codebook - cb_hi
    cb_mid = rem.astype(bf16).astype(f32)
    cb_lo = rem - cb_mid
    cb_t = codebook.T
    dec, zq, loss_parts = pl.pallas_call(
        _vqvae_body,
        grid=(_NB,),
        in_specs=[
            pl.BlockSpec((_BS, _DIN), lambda i: (i, 0)),
            _full((_DIN, _H)), _full((1, _H)),
            _full((_H, _H)), _full((1, _H)),
            _full((_H, _DC)), _full((1, _DC)),
            _full((_K, _DC)), _full((_K, _DC)), _full((_K, _DC)), _full((_DC, _K)),
            _full((_DC, _H)), _full((1, _H)),
            _full((_H, _H)), _full((1, _H)),
            _full((_H, _DIN)), _full((1, _DIN)),
        ],
        out_specs=[
            pl.BlockSpec((_BS, _DIN), lambda i: (i, 0)),
            pl.BlockSpec((_BS, _DC), lambda i: (i, 0)),
            pl.BlockSpec((1, 8, 128), lambda i: (i, 0, 0)),
        ],
        out_shape=[
            jax.ShapeDtypeStruct((_B, _DIN), jnp.float32),
            jax.ShapeDtypeStruct((_B, _DC), jnp.float32),
            jax.ShapeDtypeStruct((_NB, 8, 128), jnp.float32),
        ],
        compiler_params=pltpu.CompilerParams(
            dimension_semantics=("arbitrary",),
        ),
    )(x, enc_w1, enc_b1, enc_w2, enc_b2, enc_w3, enc_b3,
      cb_hi, cb_mid, cb_lo, cb_t,
      dec_w1, dec_b1, dec_w2, dec_b2, dec_w3, dec_b3)
    loss = jnp.sum(loss_parts[:, 0, 0]) / (_B * _DC)
    return dec, zq, loss, loss


def kernel(x, enc_w1, enc_b1, enc_w2, enc_b2, enc_w3, enc_b3, codebook,
           dec_w1, dec_b1, dec_w2, dec_b2, dec_w3, dec_b3):
    return _vqvae_fused(
        x, enc_w1, enc_b1.reshape(1, -1), enc_w2, enc_b2.reshape(1, -1),
        enc_w3, enc_b3.reshape(1, -1), codebook,
        dec_w1, dec_b1.reshape(1, -1), dec_w2, dec_b2.reshape(1, -1),
        dec_w3, dec_b3.reshape(1, -1))


# speedup vs baseline: 1.2380x; 1.0192x over previous
import functools

import jax
import jax.numpy as jnp
from jax import lax
from jax.experimental import pallas as pl
from jax.experimental.pallas import tpu as pltpu
from jax.experimental.pallas import tpu_sc as plsc

_B, _DIN, _H, _DC, _K = 16384, 512, 256, 32, 1024
_BS = 1024
_NB = _B // _BS

_NC, _NS = 2, 16
_NW = _NC * _NS
_BPW = _B // _NW
_CH = 128
_NCHUNK = _BPW // _CH


def _enc_body(x_ref, ew1, eb1, ew2, eb2, ew3, eb3, cbt,
              z_ref, idx_ref):
    f32 = jnp.float32
    x = x_ref[...]
    h = jnp.maximum(jnp.dot(x, ew1[...], preferred_element_type=f32) + eb1[...], 0.0)
    h = jnp.maximum(jnp.dot(h, ew2[...], preferred_element_type=f32) + eb2[...], 0.0)
    z = jnp.dot(h, ew3[...], preferred_element_type=f32) + eb3[...]

    cbt_v = cbt[...]
    csq = jnp.sum(cbt_v * cbt_v, axis=0, keepdims=True)
    zsq = jnp.sum(z * z, axis=1, keepdims=True)
    d = (zsq - 2.0 * jnp.dot(z, cbt_v, preferred_element_type=f32)) + csq

    dmin = jnp.min(d, axis=1, keepdims=True)
    iota_k = lax.broadcasted_iota(jnp.int32, (_BS, _K), 1).astype(f32)
    idx = jnp.min(jnp.where(d == dmin, iota_k, float(_K)), axis=1, keepdims=True)

    z_ref[...] = z
    idx_ref[...] = idx


def _dec_body(z_ref, e_ref, dw1, db1, dw2, db2, dw3, db3,
              dec_ref, zq_ref, loss_ref):
    f32 = jnp.float32
    z = z_ref[...]
    e = e_ref[...]
    zq = z + (e - z)
    zq_ref[...] = zq
    diff = z - e
    loss_ref[...] = jnp.full((1, 8, 128), jnp.sum(diff * diff), dtype=f32)

    g = jnp.maximum(jnp.dot(zq, dw1[...], preferred_element_type=f32) + db1[...], 0.0)
    g = jnp.maximum(jnp.dot(g, dw2[...], preferred_element_type=f32) + db2[...], 0.0)
    dec_ref[...] = jnp.dot(g, dw3[...], preferred_element_type=f32) + db3[...]


def _sc_gather_body(cb_hbm, idx_hbm, out_hbm, idx_v, rows_v, sem):
    wid = lax.axis_index("s") * _NC + lax.axis_index("c")
    base = wid * _BPW
    pltpu.sync_copy(idx_hbm.at[wid], idx_v)
    copies = []
    for j in range(_NCHUNK):
        copies.append(pltpu.async_copy(cb_hbm.at[idx_v.at[j]], rows_v.at[j], sem))
    for c in copies:
        c.wait()
    for j in range(_NCHUNK):
        pltpu.sync_copy(rows_v.at[j], out_hbm.at[pl.ds(base + j * _CH, _CH)])


_sc_gather = functools.partial(
    pl.kernel,
    out_type=jax.ShapeDtypeStruct((_B, _DC), jnp.float32),
    mesh=plsc.VectorSubcoreMesh(core_axis_name="c", subcore_axis_name="s"),
    scratch_types=[
        pltpu.VMEM((_NCHUNK, _CH), jnp.int32),
        pltpu.VMEM((_NCHUNK, _CH, _DC), jnp.float32),
        pltpu.SemaphoreType.DMA,
    ],
    compiler_params=pltpu.CompilerParams(use_tc_tiling_on_sc=False),
)(_sc_gather_body)


def _full(shape):
    return pl.BlockSpec(shape, lambda i: (0,) * len(shape))


@jax.jit
def _vqvae(x, enc_w1, enc_b1, enc_w2, enc_b2, enc_w3, enc_b3,
           codebook, dec_w1, dec_b1, dec_w2, dec_b2, dec_w3, dec_b3):
    z, idxf = pl.pallas_call(
        _enc_body,
        grid=(_NB,),
        in_specs=[
            pl.BlockSpec((_BS, _DIN), lambda i: (i, 0)),
            _full((_DIN, _H)), _full((1, _H)),
            _full((_H, _H)), _full((1, _H)),
            _full((_H, _DC)), _full((1, _DC)),
            _full((_DC, _K)),
        ],
        out_specs=[
            pl.BlockSpec((_BS, _DC), lambda i: (i, 0)),
            pl.BlockSpec((_BS, 1), lambda i: (i, 0)),
        ],
        out_shape=[
            jax.ShapeDtypeStruct((_B, _DC), jnp.float32),
            jax.ShapeDtypeStruct((_B, 1), jnp.float32),
        ],
        compiler_params=pltpu.CompilerParams(
            dimension_semantics=("arbitrary",),
        ),
    )(x, enc_w1, enc_b1, enc_w2, enc_b2, enc_w3, enc_b3, codebook.T)

    idx = idxf.astype(jnp.int32).reshape(_NW, _NCHUNK, _CH)
    e = _sc_gather(codebook, idx)

    dec, zq, loss_parts = pl.pallas_call(
        _dec_body,
        grid=(_NB,),
        in_specs=[
            pl.BlockSpec((_BS, _DC), lambda i: (i, 0)),
            pl.BlockSpec((_BS, _DC), lambda i: (i, 0)),
            _full((_DC, _H)), _full((1, _H)),
            _full((_H, _H)), _full((1, _H)),
            _full((_H, _DIN)), _full((1, _DIN)),
        ],
        out_specs=[
            pl.BlockSpec((_BS, _DIN), lambda i: (i, 0)),
            pl.BlockSpec((_BS, _DC), lambda i: (i, 0)),
            pl.BlockSpec((1, 8, 128), lambda i: (i, 0, 0)),
        ],
        out_shape=[
            jax.ShapeDtypeStruct((_B, _DIN), jnp.float32),
            jax.ShapeDtypeStruct((_B, _DC), jnp.float32),
            jax.ShapeDtypeStruct((_NB, 8, 128), jnp.float32),
        ],
        compiler_params=pltpu.CompilerParams(
            dimension_semantics=("arbitrary",),
        ),
    )(z, e, dec_w1, dec_b1, dec_w2, dec_b2, dec_w3, dec_b3)

    loss = jnp.sum(loss_parts[:, 0, 0]) / (_B * _DC)
    return dec, zq, loss, loss


def kernel(x, enc_w1, enc_b1, enc_w2, enc_b2, enc_w3, enc_b3, codebook,
           dec_w1, dec_b1, dec_w2, dec_b2, dec_w3, dec_b3):
    return _vqvae(
        x, enc_w1, enc_b1.reshape(1, -1), enc_w2, enc_b2.reshape(1, -1),
        enc_w3, enc_b3.reshape(1, -1), codebook,
        dec_w1, dec_b1.reshape(1, -1), dec_w2, dec_b2.reshape(1, -1),
        dec_w3, dec_b3.reshape(1, -1))


# speedup vs baseline: 1.7789x; 1.4370x over previous
import functools

import jax
import jax.numpy as jnp
from jax import lax
from jax.experimental import pallas as pl
from jax.experimental.pallas import tpu as pltpu

_B, _DIN, _H, _DC, _K = 16384, 512, 256, 32, 1024
_PREC = None
_BS = 1024
_NB = _B // _BS


def _vqvae_body(x_ref, ew1, eb1, ew2, eb2, ew3, eb3, cb3, cbt,
                dw1, db1, dw2, db2, dw3, db3,
                dec_ref, zq_ref, loss_ref):
    f32 = jnp.float32
    x = x_ref[...]
    h = jnp.maximum(jnp.dot(x, ew1[...], precision=_PREC, preferred_element_type=f32) + eb1[...], 0.0)
    h = jnp.maximum(jnp.dot(h, ew2[...], precision=_PREC, preferred_element_type=f32) + eb2[...], 0.0)
    z = jnp.dot(h, ew3[...], precision=_PREC, preferred_element_type=f32) + eb3[...]

    cbt_v = cbt[...]
    csq = jnp.sum(cbt_v * cbt_v, axis=0, keepdims=True)
    zsq = jnp.sum(z * z, axis=1, keepdims=True)
    d = (zsq - 2.0 * jnp.dot(z, cbt_v, precision=_PREC, preferred_element_type=f32)) + csq

    dmin = jnp.min(d, axis=1, keepdims=True)
    iota_k = lax.broadcasted_iota(jnp.int32, (_BS, _K), 1).astype(f32)
    idx = jnp.min(jnp.where(d == dmin, iota_k, float(_K)), axis=1, keepdims=True)
    onehot = (iota_k == idx).astype(jnp.bfloat16)
    e3 = jnp.dot(onehot, cb3[...], preferred_element_type=f32)
    e = ((e3[:, :_DC] + e3[:, _DC:2 * _DC]) + e3[:, 2 * _DC:])

    zq = z + (e - z)
    zq_ref[...] = zq
    diff = z - e
    loss_ref[...] = jnp.full((1, 8, 128), jnp.sum(diff * diff), dtype=f32)

    g = jnp.maximum(jnp.dot(e, dw1[...], precision=_PREC, preferred_element_type=f32) + db1[...], 0.0)
    g = jnp.maximum(jnp.dot(g, dw2[...], precision=_PREC, preferred_element_type=f32) + db2[...], 0.0)
    dec_ref[...] = jnp.dot(g, dw3[...], precision=_PREC, preferred_element_type=f32) + db3[...]


def _full(shape):
    return pl.BlockSpec(shape, lambda i: (0,) * len(shape))


@jax.jit
def _vqvae_fused(x, enc_w1, enc_b1, enc_w2, enc_b2, enc_w3, enc_b3,
                 codebook, dec_w1, dec_b1, dec_w2, dec_b2, dec_w3, dec_b3):
    bf16, f32 = jnp.bfloat16, jnp.float32
    cb_hi = codebook.astype(bf16)
    rem = codebook - cb_hi.astype(f32)
    cb_mid = rem.astype(bf16)
    cb_lo = (rem - cb_mid.astype(f32)).astype(bf16)
    cb3 = jnp.concatenate([cb_hi, cb_mid, cb_lo], axis=1)
    cb_t = codebook.T
    dec, zq, loss_parts = pl.pallas_call(
        _vqvae_body,
        grid=(_NB,),
        in_specs=[
            pl.BlockSpec((_BS, _DIN), lambda i: (i, 0)),
            _full((_DIN, _H)), _full((1, _H)),
            _full((_H, _H)), _full((1, _H)),
            _full((_H, _DC)), _full((1, _DC)),
            _full((_K, 3 * _DC)), _full((_DC, _K)),
            _full((_DC, _H)), _full((1, _H)),
            _full((_H, _H)), _full((1, _H)),
            _full((_H, _DIN)), _full((1, _DIN)),
        ],
        out_specs=[
            pl.BlockSpec((_BS, _DIN), lambda i: (i, 0)),
            pl.BlockSpec((_BS, _DC), lambda i: (i, 0)),
            pl.BlockSpec((1, 8, 128), lambda i: (i, 0, 0)),
        ],
        out_shape=[
            jax.ShapeDtypeStruct((_B, _DIN), jnp.float32),
            jax.ShapeDtypeStruct((_B, _DC), jnp.float32),
            jax.ShapeDtypeStruct((_NB, 8, 128), jnp.float32),
        ],
        compiler_params=pltpu.CompilerParams(
            dimension_semantics=("arbitrary",),
        ),
    )(x, enc_w1, enc_b1, enc_w2, enc_b2, enc_w3, enc_b3, cb3, cb_t,
      dec_w1, dec_b1, dec_w2, dec_b2, dec_w3, dec_b3)
    loss = jnp.sum(loss_parts[:, 0, 0]) / (_B * _DC)
    return dec, zq, loss, loss


def kernel(x, enc_w1, enc_b1, enc_w2, enc_b2, enc_w3, enc_b3, codebook,
           dec_w1, dec_b1, dec_w2, dec_b2, dec_w3, dec_b3):
    return _vqvae_fused(
        x, enc_w1, enc_b1.reshape(1, -1), enc_w2, enc_b2.reshape(1, -1),
        enc_w3, enc_b3.reshape(1, -1), codebook,
        dec_w1, dec_b1.reshape(1, -1), dec_w2, dec_b2.reshape(1, -1),
        dec_w3, dec_b3.reshape(1, -1))


# speedup vs baseline: 1.8964x; 1.0660x over previous
import functools

import jax
import jax.numpy as jnp
from jax import lax
from jax.experimental import pallas as pl
from jax.experimental.pallas import tpu as pltpu

_B, _DIN, _H, _DC, _K = 16384, 512, 256, 32, 1024
_PREC = None
_BS = 2048
_NB = _B // _BS


def _vqvae_body(x_ref, ew1, eb1, ew2, eb2, ew3, eb3, cb3, cbt,
                dw1, db1, dw2, db2, dw3, db3,
                dec_ref, zq_ref, loss_ref):
    f32 = jnp.float32
    x = x_ref[...]
    h = jnp.maximum(jnp.dot(x, ew1[...], precision=_PREC, preferred_element_type=f32) + eb1[...], 0.0)
    h = jnp.maximum(jnp.dot(h, ew2[...], precision=_PREC, preferred_element_type=f32) + eb2[...], 0.0)
    z = jnp.dot(h, ew3[...], precision=_PREC, preferred_element_type=f32) + eb3[...]

    cbt_v = cbt[...]
    csq = jnp.sum(cbt_v * cbt_v, axis=0, keepdims=True)
    zsq = jnp.sum(z * z, axis=1, keepdims=True)
    d = (zsq - 2.0 * jnp.dot(z, cbt_v, precision=_PREC, preferred_element_type=f32)) + csq

    dmin = jnp.min(d, axis=1, keepdims=True)
    iota_k = lax.broadcasted_iota(jnp.int32, (_BS, _K), 1).astype(f32)
    idx = jnp.min(jnp.where(d == dmin, iota_k, float(_K)), axis=1, keepdims=True)
    onehot = (iota_k == idx).astype(jnp.bfloat16)
    e3 = jnp.dot(onehot, cb3[...], preferred_element_type=f32)
    e = ((e3[:, :_DC] + e3[:, _DC:2 * _DC]) + e3[:, 2 * _DC:])

    zq = z + (e - z)
    zq_ref[...] = zq
    diff = z - e
    loss_ref[...] = jnp.full((1, 8, 128), jnp.sum(diff * diff), dtype=f32)

    g = jnp.maximum(jnp.dot(e, dw1[...], precision=_PREC, preferred_element_type=f32) + db1[...], 0.0)
    g = jnp.maximum(jnp.dot(g, dw2[...], precision=_PREC, preferred_element_type=f32) + db2[...], 0.0)
    dec_ref[...] = jnp.dot(g, dw3[...], precision=_PREC, preferred_element_type=f32) + db3[...]


def _full(shape):
    return pl.BlockSpec(shape, lambda i: (0,) * len(shape))


@jax.jit
def _vqvae_fused(x, enc_w1, enc_b1, enc_w2, enc_b2, enc_w3, enc_b3,
                 codebook, dec_w1, dec_b1, dec_w2, dec_b2, dec_w3, dec_b3):
    bf16, f32 = jnp.bfloat16, jnp.float32
    cb_hi = codebook.astype(bf16)
    rem = codebook - cb_hi.astype(f32)
    cb_mid = rem.astype(bf16)
    cb_lo = (rem - cb_mid.astype(f32)).astype(bf16)
    cb3 = jnp.concatenate([cb_hi, cb_mid, cb_lo], axis=1)
    cb_t = codebook.T
    dec, zq, loss_parts = pl.pallas_call(
        _vqvae_body,
        grid=(_NB,),
        in_specs=[
            pl.BlockSpec((_BS, _DIN), lambda i: (i, 0)),
            _full((_DIN, _H)), _full((1, _H)),
            _full((_H, _H)), _full((1, _H)),
            _full((_H, _DC)), _full((1, _DC)),
            _full((_K, 3 * _DC)), _full((_DC, _K)),
            _full((_DC, _H)), _full((1, _H)),
            _full((_H, _H)), _full((1, _H)),
            _full((_H, _DIN)), _full((1, _DIN)),
        ],
        out_specs=[
            pl.BlockSpec((_BS, _DIN), lambda i: (i, 0)),
            pl.BlockSpec((_BS, _DC), lambda i: (i, 0)),
            pl.BlockSpec((1, 8, 128), lambda i: (i, 0, 0)),
        ],
        out_shape=[
            jax.ShapeDtypeStruct((_B, _DIN), jnp.float32),
            jax.ShapeDtypeStruct((_B, _DC), jnp.float32),
            jax.ShapeDtypeStruct((_NB, 8, 128), jnp.float32),
        ],
        compiler_params=pltpu.CompilerParams(
            dimension_semantics=("arbitrary",),
        ),
    )(x, enc_w1, enc_b1, enc_w2, enc_b2, enc_w3, enc_b3, cb3, cb_t,
      dec_w1, dec_b1, dec_w2, dec_b2, dec_w3, dec_b3)
    loss = jnp.sum(loss_parts[:, 0, 0]) / (_B * _DC)
    return dec, zq, loss, loss


def kernel(x, enc_w1, enc_b1, enc_w2, enc_b2, enc_w3, enc_b3, codebook,
           dec_w1, dec_b1, dec_w2, dec_b2, dec_w3, dec_b3):
    return _vqvae_fused(
        x, enc_w1, enc_b1.reshape(1, -1), enc_w2, enc_b2.reshape(1, -1),
        enc_w3, enc_b3.reshape(1, -1), codebook,
        dec_w1, dec_b1.reshape(1, -1), dec_w2, dec_b2.reshape(1, -1),
        dec_w3, dec_b3.reshape(1, -1))


# speedup vs baseline: 1.9108x; 1.0076x over previous
import functools

import jax
import jax.numpy as jnp
from jax import lax
from jax.experimental import pallas as pl
from jax.experimental.pallas import tpu as pltpu

_B, _DIN, _H, _DC, _K = 16384, 512, 256, 32, 1024
_PREC = None
_BS = 4096
_NB = _B // _BS


def _vqvae_body(x_ref, ew1, eb1, ew2, eb2, ew3, eb3, cb3, cbt,
                dw1, db1, dw2, db2, dw3, db3,
                dec_ref, zq_ref, loss_ref):
    f32 = jnp.float32
    x = x_ref[...]
    h = jnp.maximum(jnp.dot(x, ew1[...], precision=_PREC, preferred_element_type=f32) + eb1[...], 0.0)
    h = jnp.maximum(jnp.dot(h, ew2[...], precision=_PREC, preferred_element_type=f32) + eb2[...], 0.0)
    z = jnp.dot(h, ew3[...], precision=_PREC, preferred_element_type=f32) + eb3[...]

    cbt_v = cbt[...]
    csq = jnp.sum(cbt_v * cbt_v, axis=0, keepdims=True)
    zsq = jnp.sum(z * z, axis=1, keepdims=True)
    d = (zsq - 2.0 * jnp.dot(z, cbt_v, precision=_PREC, preferred_element_type=f32)) + csq

    dmin = jnp.min(d, axis=1, keepdims=True)
    iota_k = lax.broadcasted_iota(jnp.int32, (_BS, _K), 1).astype(f32)
    idx = jnp.min(jnp.where(d == dmin, iota_k, float(_K)), axis=1, keepdims=True)
    onehot = (iota_k == idx).astype(jnp.bfloat16)
    e3 = jnp.dot(onehot, cb3[...], preferred_element_type=f32)
    e = ((e3[:, :_DC] + e3[:, _DC:2 * _DC]) + e3[:, 2 * _DC:])

    zq = z + (e - z)
    zq_ref[...] = zq
    diff = z - e
    loss_ref[...] = jnp.full((1, 8, 128), jnp.sum(diff * diff), dtype=f32)

    g = jnp.maximum(jnp.dot(e, dw1[...], precision=_PREC, preferred_element_type=f32) + db1[...], 0.0)
    g = jnp.maximum(jnp.dot(g, dw2[...], precision=_PREC, preferred_element_type=f32) + db2[...], 0.0)
    dec_ref[...] = jnp.dot(g, dw3[...], precision=_PREC, preferred_element_type=f32) + db3[...]


def _full(shape):
    return pl.BlockSpec(shape, lambda i: (0,) * len(shape))


@jax.jit
def _vqvae_fused(x, enc_w1, enc_b1, enc_w2, enc_b2, enc_w3, enc_b3,
                 codebook, dec_w1, dec_b1, dec_w2, dec_b2, dec_w3, dec_b3):
    bf16, f32 = jnp.bfloat16, jnp.float32
    cb_hi = codebook.astype(bf16)
    rem = codebook - cb_hi.astype(f32)
    cb_mid = rem.astype(bf16)
    cb_lo = (rem - cb_mid.astype(f32)).astype(bf16)
    cb3 = jnp.concatenate([cb_hi, cb_mid, cb_lo], axis=1)
    cb_t = codebook.T
    dec, zq, loss_parts = pl.pallas_call(
        _vqvae_body,
        grid=(_NB,),
        in_specs=[
            pl.BlockSpec((_BS, _DIN), lambda i: (i, 0)),
            _full((_DIN, _H)), _full((1, _H)),
            _full((_H, _H)), _full((1, _H)),
            _full((_H, _DC)), _full((1, _DC)),
            _full((_K, 3 * _DC)), _full((_DC, _K)),
            _full((_DC, _H)), _full((1, _H)),
            _full((_H, _H)), _full((1, _H)),
            _full((_H, _DIN)), _full((1, _DIN)),
        ],
        out_specs=[
            pl.BlockSpec((_BS, _DIN), lambda i: (i, 0)),
            pl.BlockSpec((_BS, _DC), lambda i: (i, 0)),
            pl.BlockSpec((1, 8, 128), lambda i: (i, 0, 0)),
        ],
        out_shape=[
            jax.ShapeDtypeStruct((_B, _DIN), jnp.float32),
            jax.ShapeDtypeStruct((_B, _DC), jnp.float32),
            jax.ShapeDtypeStruct((_NB, 8, 128), jnp.float32),
        ],
        compiler_params=pltpu.CompilerParams(
            dimension_semantics=("arbitrary",),
        ),
    )(x, enc_w1, enc_b1, enc_w2, enc_b2, enc_w3, enc_b3, cb3, cb_t,
      dec_w1, dec_b1, dec_w2, dec_b2, dec_w3, dec_b3)
    loss = jnp.sum(loss_parts[:, 0, 0]) / (_B * _DC)
    return dec, zq, loss, loss


def kernel(x, enc_w1, enc_b1, enc_w2, enc_b2, enc_w3, enc_b3, codebook,
           dec_w1, dec_b1, dec_w2, dec_b2, dec_w3, dec_b3):
    return _vqvae_fused(
        x, enc_w1, enc_b1.reshape(1, -1), enc_w2, enc_b2.reshape(1, -1),
        enc_w3, enc_b3.reshape(1, -1), codebook,
        dec_w1, dec_b1.reshape(1, -1), dec_w2, dec_b2.reshape(1, -1),
        dec_w3, dec_b3.reshape(1, -1))
